# baseline (speedup 1.0000x reference)
import functools

import jax
import jax.numpy as jnp
from jax import lax
from jax.experimental import pallas as pl
from jax.experimental.pallas import tpu as pltpu
from jax.experimental.pallas import tpu_sc as plsc

EMB_DIM = 64
N_OUTER = 16384
N_INNER = 50
B_TOTAL = N_OUTER * N_INNER

_info = plsc.get_sparse_core_info()
_NC, _NS = _info.num_cores, _info.num_subcores
_NW = _NC * _NS
_B_PER_W = B_TOTAL // _NW
_NBUF = 3
_CHUNK = 512
_NCHUNK = _B_PER_W // _CHUNK

_mesh = plsc.VectorSubcoreMesh(core_axis_name="c", subcore_axis_name="s")


@functools.partial(
    pl.kernel,
    mesh=_mesh,
    out_type=jax.ShapeDtypeStruct((N_INNER, N_OUTER, EMB_DIM), jnp.float32),
    scratch_types=[
        pltpu.VMEM((_B_PER_W,), jnp.int32),
        pltpu.VMEM((_NBUF, _CHUNK, EMB_DIM), jnp.float32),
        pltpu.SemaphoreType.DMA,
        pltpu.SemaphoreType.DMA((_NBUF,)),
        pltpu.SemaphoreType.DMA((_NBUF,)),
    ],
    compiler_params=pltpu.CompilerParams(use_tc_tiling_on_sc=False),
)
def _gather_kernel(xt_hbm, table_hbm, out_hbm, idx_v, rows_v, isem, gsem,
                   wsem):
    wid = lax.axis_index("s") * _NC + lax.axis_index("c")
    base = wid * _B_PER_W
    pltpu.async_copy(xt_hbm.at[pl.ds(base, _B_PER_W)], idx_v, isem)
    pltpu.make_async_copy(xt_hbm.at[pl.ds(0, _B_PER_W)], idx_v, isem).wait()

    def start_gather(i, b):
        pltpu.async_copy(
            table_hbm.at[idx_v.at[pl.ds(i * _CHUNK, _CHUNK)]],
            rows_v.at[b], gsem.at[b])

    def wait_gather(b):
        pltpu.make_async_copy(
            table_hbm.at[idx_v.at[pl.ds(0, _CHUNK)]],
            rows_v.at[b], gsem.at[b]).wait()

    def start_wb(i, b):
        n = base + i * _CHUNK
        pltpu.async_copy(
            rows_v.at[b],
            out_hbm.at[n // N_OUTER, pl.ds(n % N_OUTER, _CHUNK)],
            wsem.at[b])

    def wait_wb(b):
        pltpu.make_async_copy(
            table_hbm.at[pl.ds(0, _CHUNK)], rows_v.at[b], wsem.at[b]).wait()

    start_gather(0, 0)

    def step(m, _):
        for u in range(_NBUF):
            s = m * _NBUF + u
            b = u
            b1 = (u + 1) % _NBUF

            @pl.when(s + 1 < _NCHUNK)
            def _():
                @pl.when(s + 1 >= _NBUF)
                def _():
                    wait_wb(b1)

                start_gather(s + 1, b1)

            @pl.when(s < _NCHUNK)
            def _():
                wait_gather(b)
                start_wb(s, b)

        return ()

    nsteps = (_NCHUNK + _NBUF - 1) // _NBUF
    lax.fori_loop(0, nsteps, step, ())

    for b in range(_NBUF):
        wait_wb(b)


def kernel(x, table):
    xt = x.T.astype(jnp.int32).reshape(-1)
    outj = _gather_kernel(xt, table)
    return outj.transpose(1, 0, 2)
